# baseline (speedup 1.0000x reference)
import functools

import jax
import jax.numpy as jnp
from jax import lax
from jax.experimental import pallas as pl
from jax.experimental.pallas import tpu as pltpu
from jax.experimental.pallas import tpu_sc as plsc

N = 10000
NP = 10240
E = 160000
NC = 2
NS = 16
NW = NC * NS
EW = E // NW
RW = NP // NS
ES = 163840
ET = ES // NS
ACC_R = 10016
JUNK = N
RT = N // NS
F1 = 256
F2 = 64
RB = 1000

_mesh = plsc.VectorSubcoreMesh(
    core_axis_name="c", subcore_axis_name="s", num_cores=NC, num_subcores=NS
)

_sc_params = pltpu.CompilerParams(use_tc_tiling_on_sc=False)


@functools.partial(
    pl.kernel,
    out_type=jax.ShapeDtypeStruct((NC * NP,), jnp.float32),
    mesh=_mesh,
    scratch_types=[
        pltpu.VMEM((EW,), jnp.int32),
        pltpu.VMEM((EW + 8,), jnp.float32),
        pltpu.VMEM((RW,), jnp.float32),
        pltpu.VMEM_SHARED((NP,), jnp.float32),
    ],
    compiler_params=_sc_params,
)
def _sc_degree(dst_hbm, out_hbm, idx_v, ones_v, zeros_v, deg_sh):
    c = lax.axis_index("c")
    s = lax.axis_index("s")
    w = s * NC + c

    @pl.loop(0, EW + 8, step=16)
    def _(i):
        ones_v[pl.ds(i, 16)] = jnp.full((16,), 1.0, jnp.float32)

    @pl.loop(0, RW, step=16)
    def _(i):
        zeros_v[pl.ds(i, 16)] = jnp.zeros((16,), jnp.float32)

    pltpu.sync_copy(zeros_v, deg_sh.at[pl.ds(s * RW, RW)])
    plsc.subcore_barrier()
    pltpu.sync_copy(dst_hbm.at[pl.ds(w * EW, EW)], idx_v)
    pltpu.sync_copy(ones_v.at[pl.ds(0, EW)], deg_sh.at[idx_v], add=True)
    plsc.subcore_barrier()
    pltpu.sync_copy(
        deg_sh.at[pl.ds(s * RW, RW)], out_hbm.at[pl.ds(c * NP + s * RW, RW)]
    )


def _make_sc_aggregate(fc, ke):
    nwin = ET // ke
    nwt = ES // ke
    assert nwin % 2 == 0 and nwin >= 4 and ke % 8 == 0 and ET % ke == 0

    @functools.partial(
        pl.kernel,
        out_type=jax.ShapeDtypeStruct((NC * N, fc), jnp.float32),
        mesh=_mesh,
        scratch_types=[
            pltpu.VMEM((2, ke), jnp.int32),
            pltpu.VMEM((2, ke), jnp.int32),
            pltpu.VMEM((ke, fc), jnp.float32),
            pltpu.VMEM((ke, fc), jnp.float32),
            pltpu.VMEM_SHARED((ACC_R, fc), jnp.float32),
            pltpu.SemaphoreType.DMA,
            pltpu.SemaphoreType.DMA,
            pltpu.SemaphoreType.DMA,
            pltpu.SemaphoreType.DMA,
            pltpu.SemaphoreType.DMA,
        ],
        compiler_params=_sc_params,
    )
    def agg(y_hbm, sd_hbm, out_hbm, sd_a, sd_b, rows_a, rows_b, acc_sh,
            sem_la, sem_lb, sem_ga, sem_gb, sem_i):
        c = lax.axis_index("c")
        s = lax.axis_index("s")
        row0 = (c * nwt + s * nwin) * 2

        init = pltpu.async_copy(
            y_hbm.at[pl.ds(c * N + s * RT, RT)],
            acc_sh.at[pl.ds(s * RT, RT)],
            sem_i,
        )
        pltpu.sync_copy(sd_hbm.at[pl.ds(row0, 2), :], sd_a)
        pltpu.async_copy(sd_hbm.at[pl.ds(row0 + 2, 2), :], sd_b, sem_lb)
        pltpu.async_copy(y_hbm.at[sd_a.at[0]], rows_a, sem_ga)
        init.wait()
        plsc.subcore_barrier()

        @pl.loop(0, nwin, step=2)
        def _(j):
            pltpu.make_async_copy(sd_hbm.at[pl.ds(0, 2), :], sd_b, sem_lb).wait()
            gb = pltpu.async_copy(y_hbm.at[sd_b.at[0]], rows_b, sem_gb)
            pltpu.make_async_copy(y_hbm.at[pl.ds(0, ke)], rows_a, sem_ga).wait()
            pltpu.sync_copy(rows_a, acc_sh.at[sd_a.at[1]], add=True)

            @pl.when(j + 2 < nwin)
            def _():
                pltpu.async_copy(
                    sd_hbm.at[pl.ds(row0 + (j + 2) * 2, 2), :], sd_a, sem_la
                )

            gb.wait()
            pltpu.sync_copy(rows_b, acc_sh.at[sd_b.at[1]], add=True)

            @pl.when(j + 2 < nwin)
            def _():
                pltpu.make_async_copy(sd_hbm.at[pl.ds(0, 2), :], sd_a, sem_la).wait()
                pltpu.async_copy(y_hbm.at[sd_a.at[0]], rows_a, sem_ga)

            @pl.when(j + 3 < nwin)
            def _():
                pltpu.async_copy(
                    sd_hbm.at[pl.ds(row0 + (j + 3) * 2, 2), :], sd_b, sem_lb
                )

        plsc.subcore_barrier()
        pltpu.sync_copy(
            acc_sh.at[pl.ds(s * RT, RT)],
            out_hbm.at[pl.ds(c * N + s * RT, RT), :],
        )

    return agg


KE1 = 160
KE2 = 640
_sc_agg1 = _make_sc_aggregate(F1 // 2, KE1)
_sc_agg2 = _make_sc_aggregate(F2 // 2, KE2)


def _make_sd(src, dst, ke):
    pad_s = jnp.zeros((ES - E,), jnp.int32)
    pad_d = jnp.full((ES - E,), JUNK, jnp.int32)
    sw = jnp.concatenate([src, pad_s]).reshape(ES // ke, ke)
    dw = jnp.concatenate([dst, pad_d]).reshape(ES // ke, ke)
    per_core = [
        jnp.stack([sw + c * N, dw], axis=1) for c in range(NC)
    ]
    return jnp.concatenate(per_core).reshape(NC * (ES // ke) * 2, ke)


def _dinv(deg_ref):
    return lax.rsqrt(deg_ref[:, 0] + deg_ref[:, 1] + 1.0)[:, None]


def _dot(a, b):
    return jax.lax.dot(
        a, b, precision=jax.lax.Precision.HIGHEST,
        preferred_element_type=jnp.float32,
    )


def _mm1_body(x_ref, w_ref, o_ref):
    o_ref[...] = _dot(x_ref[...], w_ref[...])


def _tc_matmul1(x, w1):
    return pl.pallas_call(
        _mm1_body,
        grid=(N // RB,),
        in_specs=[
            pl.BlockSpec((RB, F1), lambda i: (i, 0)),
            pl.BlockSpec((F1, F1), lambda i: (0, 0)),
        ],
        out_specs=pl.BlockSpec((RB, F1), lambda i: (i, 0)),
        out_shape=jax.ShapeDtypeStruct((N, F1), jnp.float32),
    )(x, w1)


def _scale_body(p_ref, deg_ref, o_ref):
    d = _dinv(deg_ref)
    o_ref[0] = p_ref[:, : F1 // 2] * d
    o_ref[1] = p_ref[:, F1 // 2 :] * d


def _tc_scale_split(p, deg2):
    return pl.pallas_call(
        _scale_body,
        grid=(N // RB,),
        in_specs=[
            pl.BlockSpec((RB, F1), lambda i: (i, 0)),
            pl.BlockSpec((RB, NC), lambda i: (i, 0)),
        ],
        out_specs=pl.BlockSpec((NC, RB, F1 // 2), lambda i: (0, i, 0)),
        out_shape=jax.ShapeDtypeStruct((NC, N, F1 // 2), jnp.float32),
    )(p, deg2)


def _layer2_body(a_ref, deg_ref, b1_ref, w2_ref, o_ref):
    d = _dinv(deg_ref)
    h0 = jnp.maximum(a_ref[0] * d + b1_ref[0, : F1 // 2], 0.0)
    h1 = jnp.maximum(a_ref[1] * d + b1_ref[0, F1 // 2 :], 0.0)
    y = _dot(h0, w2_ref[: F1 // 2, :]) + _dot(h1, w2_ref[F1 // 2 :, :])
    y = y * d
    o_ref[0] = y[:, : F2 // 2]
    o_ref[1] = y[:, F2 // 2 :]


def _tc_layer2(agg1, deg2, b1, w2):
    return pl.pallas_call(
        _layer2_body,
        grid=(N // RB,),
        in_specs=[
            pl.BlockSpec((NC, RB, F1 // 2), lambda i: (0, i, 0)),
            pl.BlockSpec((RB, NC), lambda i: (i, 0)),
            pl.BlockSpec((1, F1), lambda i: (0, 0)),
            pl.BlockSpec((F1, F2), lambda i: (0, 0)),
        ],
        out_specs=pl.BlockSpec((NC, RB, F2 // 2), lambda i: (0, i, 0)),
        out_shape=jax.ShapeDtypeStruct((NC, N, F2 // 2), jnp.float32),
    )(agg1, deg2, b1, w2)


def _final_body(a_ref, deg_ref, b2_ref, o_ref):
    d = _dinv(deg_ref)
    z = jnp.concatenate([a_ref[0], a_ref[1]], axis=1) * d + b2_ref[0, :]
    m = jnp.max(z, axis=1, keepdims=True)
    e = z - m
    lse = jnp.log(jnp.sum(jnp.exp(e), axis=1, keepdims=True))
    o_ref[...] = e - lse


def _tc_final(agg2, deg2, b2):
    return pl.pallas_call(
        _final_body,
        grid=(N // RB,),
        in_specs=[
            pl.BlockSpec((NC, RB, F2 // 2), lambda i: (0, i, 0)),
            pl.BlockSpec((RB, NC), lambda i: (i, 0)),
            pl.BlockSpec((1, F2), lambda i: (0, 0)),
        ],
        out_specs=pl.BlockSpec((RB, F2), lambda i: (i, 0)),
        out_shape=jax.ShapeDtypeStruct((N, F2), jnp.float32),
    )(agg2, deg2, b2)


@jax.jit
def kernel(X, edge_index, W1, b1, W2, b2):
    src = edge_index[0]
    dst = edge_index[1]
    sd1 = _make_sd(src, dst, KE1)
    sd2 = _make_sd(src, dst, KE2)

    deg2 = _sc_degree(dst).reshape(NC, NP)[:, :N].T
    p = _tc_matmul1(X, W1)
    y1 = _tc_scale_split(p, deg2)
    agg1 = _sc_agg1(y1.reshape(NC * N, F1 // 2), sd1).reshape(NC, N, F1 // 2)
    y2 = _tc_layer2(agg1, deg2, b1.reshape(1, F1), W2)
    agg2 = _sc_agg2(y2.reshape(NC * N, F2 // 2), sd2).reshape(NC, N, F2 // 2)
    out = _tc_final(agg2, deg2, b2.reshape(1, F2))
    return out


# speedup vs baseline: 1.8882x; 1.8882x over previous
import functools

import jax
import jax.numpy as jnp
from jax import lax
from jax.experimental import pallas as pl
from jax.experimental.pallas import tpu as pltpu
from jax.experimental.pallas import tpu_sc as plsc

N = 10000
NP = 10240
E = 160000
NC = 2
NS = 16
NW = NC * NS
EW = E // NW
RW = NP // NS
ES = 163840
ET = ES // NS
ACC_R = 10240
RT = N // NS
F1 = 256
F2 = 64
RB = 1000

_mesh = plsc.VectorSubcoreMesh(
    core_axis_name="c", subcore_axis_name="s", num_cores=NC, num_subcores=NS
)

_sc_params = pltpu.CompilerParams(use_tc_tiling_on_sc=False)


@functools.partial(
    pl.kernel,
    out_type=jax.ShapeDtypeStruct((NC * NP,), jnp.float32),
    mesh=_mesh,
    scratch_types=[
        pltpu.VMEM((EW,), jnp.int32),
        pltpu.VMEM((EW + 8,), jnp.float32),
        pltpu.VMEM((RW,), jnp.float32),
        pltpu.VMEM_SHARED((NP,), jnp.float32),
    ],
    compiler_params=_sc_params,
)
def _sc_degree(dst_hbm, out_hbm, idx_v, ones_v, zeros_v, deg_sh):
    c = lax.axis_index("c")
    s = lax.axis_index("s")
    w = s * NC + c

    @pl.loop(0, EW + 8, step=16)
    def _(i):
        ones_v[pl.ds(i, 16)] = jnp.full((16,), 1.0, jnp.float32)

    @pl.loop(0, RW, step=16)
    def _(i):
        zeros_v[pl.ds(i, 16)] = jnp.zeros((16,), jnp.float32)

    pltpu.sync_copy(zeros_v, deg_sh.at[pl.ds(s * RW, RW)])
    plsc.subcore_barrier()
    pltpu.sync_copy(dst_hbm.at[pl.ds(w * EW, EW)], idx_v)
    pltpu.sync_copy(ones_v.at[pl.ds(0, EW)], deg_sh.at[idx_v], add=True)
    plsc.subcore_barrier()
    pltpu.sync_copy(
        deg_sh.at[pl.ds(s * RW, RW)], out_hbm.at[pl.ds(c * NP + s * RW, RW)]
    )


def _make_sc_aggregate(fc, ke):
    nwin = ET // ke
    nwt = ES // ke
    assert nwin % 2 == 0 and nwin >= 4 and ke % 8 == 0 and ET % ke == 0

    @functools.partial(
        pl.kernel,
        out_type=jax.ShapeDtypeStruct((NC * N, fc), jnp.float32),
        mesh=_mesh,
        scratch_types=[
            pltpu.VMEM((2, ke), jnp.int32),
            pltpu.VMEM((2, ke), jnp.int32),
            pltpu.VMEM((ke, fc), jnp.float32),
            pltpu.VMEM((ke, fc), jnp.float32),
            pltpu.VMEM_SHARED((ACC_R, fc), jnp.float32),
            pltpu.SemaphoreType.DMA,
            pltpu.SemaphoreType.DMA,
            pltpu.SemaphoreType.DMA,
            pltpu.SemaphoreType.DMA,
            pltpu.SemaphoreType.DMA,
        ],
        compiler_params=_sc_params,
    )
    def agg(y_hbm, sd_hbm, out_hbm, sd_a, sd_b, rows_a, rows_b, acc_sh,
            sem_la, sem_lb, sem_ga, sem_gb, sem_i):
        c = lax.axis_index("c")
        s = lax.axis_index("s")
        row0 = (c * nwt + s * nwin) * 2

        init = pltpu.async_copy(
            y_hbm.at[pl.ds(c * N + s * RT, RT)],
            acc_sh.at[pl.ds(s * RT, RT)],
            sem_i,
        )
        pltpu.sync_copy(sd_hbm.at[pl.ds(row0, 2), :], sd_a)
        pltpu.async_copy(sd_hbm.at[pl.ds(row0 + 2, 2), :], sd_b, sem_lb)
        pltpu.async_copy(y_hbm.at[sd_a.at[0]], rows_a, sem_ga)
        init.wait()
        plsc.subcore_barrier()

        @pl.loop(0, nwin, step=2)
        def _(j):
            pltpu.make_async_copy(sd_hbm.at[pl.ds(0, 2), :], sd_b, sem_lb).wait()
            gb = pltpu.async_copy(y_hbm.at[sd_b.at[0]], rows_b, sem_gb)
            pltpu.make_async_copy(y_hbm.at[pl.ds(0, ke)], rows_a, sem_ga).wait()
            pltpu.sync_copy(rows_a, acc_sh.at[sd_a.at[1]], add=True)

            @pl.when(j + 2 < nwin)
            def _():
                pltpu.async_copy(
                    sd_hbm.at[pl.ds(row0 + (j + 2) * 2, 2), :], sd_a, sem_la
                )

            gb.wait()
            pltpu.sync_copy(rows_b, acc_sh.at[sd_b.at[1]], add=True)

            @pl.when(j + 2 < nwin)
            def _():
                pltpu.make_async_copy(sd_hbm.at[pl.ds(0, 2), :], sd_a, sem_la).wait()
                pltpu.async_copy(y_hbm.at[sd_a.at[0]], rows_a, sem_ga)

            @pl.when(j + 3 < nwin)
            def _():
                pltpu.async_copy(
                    sd_hbm.at[pl.ds(row0 + (j + 3) * 2, 2), :], sd_b, sem_lb
                )

        plsc.subcore_barrier()
        pltpu.sync_copy(
            acc_sh.at[pl.ds(s * RT, RT)],
            out_hbm.at[pl.ds(c * N + s * RT, RT), :],
        )

    return agg


KE1 = 160
KE2 = 640
_sc_agg1 = _make_sc_aggregate(F1 // 2, KE1)
_sc_agg2 = _make_sc_aggregate(F2 // 2, KE2)


def _make_sd(src, dst, ke):
    pad = jnp.arange(ES - E, dtype=jnp.int32)
    pad_s = (pad * 997) % N
    pad_d = N + pad % (ACC_R - N)
    sw = jnp.concatenate([src, pad_s]).reshape(ES // ke, ke)
    dw = jnp.concatenate([dst, pad_d]).reshape(ES // ke, ke)
    per_core = [
        jnp.stack([sw + c * N, dw], axis=1) for c in range(NC)
    ]
    return jnp.concatenate(per_core).reshape(NC * (ES // ke) * 2, ke)


def _dinv(deg_ref):
    return lax.rsqrt(deg_ref[:, 0] + deg_ref[:, 1] + 1.0)[:, None]


def _dot(a, b):
    return jax.lax.dot(
        a, b, precision=jax.lax.Precision.HIGHEST,
        preferred_element_type=jnp.float32,
    )


def _mm1_body(x_ref, w_ref, o_ref):
    o_ref[...] = _dot(x_ref[...], w_ref[...])


def _tc_matmul1(x, w1):
    return pl.pallas_call(
        _mm1_body,
        grid=(N // RB,),
        in_specs=[
            pl.BlockSpec((RB, F1), lambda i: (i, 0)),
            pl.BlockSpec((F1, F1), lambda i: (0, 0)),
        ],
        out_specs=pl.BlockSpec((RB, F1), lambda i: (i, 0)),
        out_shape=jax.ShapeDtypeStruct((N, F1), jnp.float32),
    )(x, w1)


def _scale_body(p_ref, deg_ref, o_ref):
    d = _dinv(deg_ref)
    o_ref[0] = p_ref[:, : F1 // 2] * d
    o_ref[1] = p_ref[:, F1 // 2 :] * d


def _tc_scale_split(p, deg2):
    return pl.pallas_call(
        _scale_body,
        grid=(N // RB,),
        in_specs=[
            pl.BlockSpec((RB, F1), lambda i: (i, 0)),
            pl.BlockSpec((RB, NC), lambda i: (i, 0)),
        ],
        out_specs=pl.BlockSpec((NC, RB, F1 // 2), lambda i: (0, i, 0)),
        out_shape=jax.ShapeDtypeStruct((NC, N, F1 // 2), jnp.float32),
    )(p, deg2)


def _layer2_body(a_ref, deg_ref, b1_ref, w2_ref, o_ref):
    d = _dinv(deg_ref)
    h0 = jnp.maximum(a_ref[0] * d + b1_ref[0, : F1 // 2], 0.0)
    h1 = jnp.maximum(a_ref[1] * d + b1_ref[0, F1 // 2 :], 0.0)
    y = _dot(h0, w2_ref[: F1 // 2, :]) + _dot(h1, w2_ref[F1 // 2 :, :])
    y = y * d
    o_ref[0] = y[:, : F2 // 2]
    o_ref[1] = y[:, F2 // 2 :]


def _tc_layer2(agg1, deg2, b1, w2):
    return pl.pallas_call(
        _layer2_body,
        grid=(N // RB,),
        in_specs=[
            pl.BlockSpec((NC, RB, F1 // 2), lambda i: (0, i, 0)),
            pl.BlockSpec((RB, NC), lambda i: (i, 0)),
            pl.BlockSpec((1, F1), lambda i: (0, 0)),
            pl.BlockSpec((F1, F2), lambda i: (0, 0)),
        ],
        out_specs=pl.BlockSpec((NC, RB, F2 // 2), lambda i: (0, i, 0)),
        out_shape=jax.ShapeDtypeStruct((NC, N, F2 // 2), jnp.float32),
    )(agg1, deg2, b1, w2)


def _final_body(a_ref, deg_ref, b2_ref, o_ref):
    d = _dinv(deg_ref)
    z = jnp.concatenate([a_ref[0], a_ref[1]], axis=1) * d + b2_ref[0, :]
    m = jnp.max(z, axis=1, keepdims=True)
    e = z - m
    lse = jnp.log(jnp.sum(jnp.exp(e), axis=1, keepdims=True))
    o_ref[...] = e - lse


def _tc_final(agg2, deg2, b2):
    return pl.pallas_call(
        _final_body,
        grid=(N // RB,),
        in_specs=[
            pl.BlockSpec((NC, RB, F2 // 2), lambda i: (0, i, 0)),
            pl.BlockSpec((RB, NC), lambda i: (i, 0)),
            pl.BlockSpec((1, F2), lambda i: (0, 0)),
        ],
        out_specs=pl.BlockSpec((RB, F2), lambda i: (i, 0)),
        out_shape=jax.ShapeDtypeStruct((N, F2), jnp.float32),
    )(agg2, deg2, b2)


@jax.jit
def kernel(X, edge_index, W1, b1, W2, b2):
    src = edge_index[0]
    dst = edge_index[1]
    sd1 = _make_sd(src, dst, KE1)
    sd2 = _make_sd(src, dst, KE2)

    deg2 = _sc_degree(dst).reshape(NC, NP)[:, :N].T
    p = _tc_matmul1(X, W1)
    y1 = _tc_scale_split(p, deg2)
    agg1 = _sc_agg1(y1.reshape(NC * N, F1 // 2), sd1).reshape(NC, N, F1 // 2)
    y2 = _tc_layer2(agg1, deg2, b1.reshape(1, F1), W2)
    agg2 = _sc_agg2(y2.reshape(NC * N, F2 // 2), sd2).reshape(NC, N, F2 // 2)
    out = _tc_final(agg2, deg2, b2.reshape(1, F2))
    return out


# speedup vs baseline: 2.0467x; 1.0839x over previous
import functools

import jax
import jax.numpy as jnp
from jax import lax
from jax.experimental import pallas as pl
from jax.experimental.pallas import tpu as pltpu
from jax.experimental.pallas import tpu_sc as plsc

N = 10000
NP = 10240
E = 160000
NC = 2
NS = 16
NW = NC * NS
EW = E // NW
RW = NP // NS
ES = 163840
ET = ES // NS
ACC_R = 10240
RT = N // NS
F1 = 256
F2 = 64
RB = 1000

_mesh = plsc.VectorSubcoreMesh(
    core_axis_name="c", subcore_axis_name="s", num_cores=NC, num_subcores=NS
)

_sc_params = pltpu.CompilerParams(use_tc_tiling_on_sc=False)


@functools.partial(
    pl.kernel,
    out_type=jax.ShapeDtypeStruct((NC * NP,), jnp.float32),
    mesh=_mesh,
    scratch_types=[
        pltpu.VMEM((EW,), jnp.int32),
        pltpu.VMEM((EW + 8,), jnp.float32),
        pltpu.VMEM((RW,), jnp.float32),
        pltpu.VMEM_SHARED((NP,), jnp.float32),
    ],
    compiler_params=_sc_params,
)
def _sc_degree(dst_hbm, out_hbm, idx_v, ones_v, zeros_v, deg_sh):
    c = lax.axis_index("c")
    s = lax.axis_index("s")
    w = s * NC + c

    @pl.loop(0, EW + 8, step=16)
    def _(i):
        ones_v[pl.ds(i, 16)] = jnp.full((16,), 1.0, jnp.float32)

    @pl.loop(0, RW, step=16)
    def _(i):
        zeros_v[pl.ds(i, 16)] = jnp.zeros((16,), jnp.float32)

    pltpu.sync_copy(zeros_v, deg_sh.at[pl.ds(s * RW, RW)])
    plsc.subcore_barrier()
    pltpu.sync_copy(dst_hbm.at[pl.ds(w * EW, EW)], idx_v)
    pltpu.sync_copy(ones_v.at[pl.ds(0, EW)], deg_sh.at[idx_v], add=True)
    plsc.subcore_barrier()
    pltpu.sync_copy(
        deg_sh.at[pl.ds(s * RW, RW)], out_hbm.at[pl.ds(c * NP + s * RW, RW)]
    )


def _make_sc_aggregate(fc, ke):
    nwin = ET // ke
    nwt = ES // ke
    assert nwin % 2 == 0 and nwin >= 4 and ke % 8 == 0 and ET % ke == 0

    @functools.partial(
        pl.kernel,
        out_type=jax.ShapeDtypeStruct((NC * N, fc), jnp.float32),
        mesh=_mesh,
        scratch_types=[
            pltpu.VMEM((2, ke), jnp.int32),
            pltpu.VMEM((2, ke), jnp.int32),
            pltpu.VMEM((ke, fc), jnp.float32),
            pltpu.VMEM((ke, fc), jnp.float32),
            pltpu.VMEM_SHARED((ACC_R, fc), jnp.float32),
            pltpu.SemaphoreType.DMA,
            pltpu.SemaphoreType.DMA,
            pltpu.SemaphoreType.DMA,
            pltpu.SemaphoreType.DMA,
            pltpu.SemaphoreType.DMA,
        ],
        compiler_params=_sc_params,
    )
    def agg(y_hbm, sd_hbm, out_hbm, sd_a, sd_b, rows_a, rows_b, acc_sh,
            sem_la, sem_lb, sem_ga, sem_gb, sem_i):
        c = lax.axis_index("c")
        s = lax.axis_index("s")
        row0 = (c * nwt + s * nwin) * 2

        init = pltpu.async_copy(
            y_hbm.at[pl.ds(c * N + s * RT, RT)],
            acc_sh.at[pl.ds(s * RT, RT)],
            sem_i,
        )
        pltpu.sync_copy(sd_hbm.at[pl.ds(row0, 2), :], sd_a)
        pltpu.async_copy(sd_hbm.at[pl.ds(row0 + 2, 2), :], sd_b, sem_lb)
        pltpu.async_copy(y_hbm.at[sd_a.at[0]], rows_a, sem_ga)
        init.wait()
        plsc.subcore_barrier()

        @pl.loop(0, nwin, step=2)
        def _(j):
            pltpu.make_async_copy(sd_hbm.at[pl.ds(0, 2), :], sd_b, sem_lb).wait()
            gb = pltpu.async_copy(y_hbm.at[sd_b.at[0]], rows_b, sem_gb)
            pltpu.make_async_copy(y_hbm.at[pl.ds(0, ke)], rows_a, sem_ga).wait()
            pltpu.sync_copy(rows_a, acc_sh.at[sd_a.at[1]], add=True)

            @pl.when(j + 2 < nwin)
            def _():
                pltpu.async_copy(
                    sd_hbm.at[pl.ds(row0 + (j + 2) * 2, 2), :], sd_a, sem_la
                )

            gb.wait()
            pltpu.sync_copy(rows_b, acc_sh.at[sd_b.at[1]], add=True)

            @pl.when(j + 2 < nwin)
            def _():
                pltpu.make_async_copy(sd_hbm.at[pl.ds(0, 2), :], sd_a, sem_la).wait()
                pltpu.async_copy(y_hbm.at[sd_a.at[0]], rows_a, sem_ga)

            @pl.when(j + 3 < nwin)
            def _():
                pltpu.async_copy(
                    sd_hbm.at[pl.ds(row0 + (j + 3) * 2, 2), :], sd_b, sem_lb
                )

        plsc.subcore_barrier()
        pltpu.sync_copy(
            acc_sh.at[pl.ds(s * RT, RT)],
            out_hbm.at[pl.ds(c * N + s * RT, RT), :],
        )

    return agg


KE1 = 160
KE2 = 640
_sc_agg1 = _make_sc_aggregate(F1 // 2, KE1)
_sc_agg2 = _make_sc_aggregate(F2 // 2, KE2)


def _make_sd(src, dst, ke):
    pad = jnp.arange(ES - E, dtype=jnp.int32)
    pad_s = (pad * 997) % N
    pad_d = N + pad % (ACC_R - N)
    sw = jnp.concatenate([src, pad_s]).reshape(ES // ke, ke)
    dw = jnp.concatenate([dst, pad_d]).reshape(ES // ke, ke)
    per_core = [
        jnp.stack([sw + c * N, dw], axis=1) for c in range(NC)
    ]
    return jnp.concatenate(per_core).reshape(NC * (ES // ke) * 2, ke)


def _dinv(deg_ref):
    return lax.rsqrt(deg_ref[:, 0] + deg_ref[:, 1] + 1.0)[:, None]


def _dot(a, b):
    return jax.lax.dot(a, b, preferred_element_type=jnp.float32)


def _mm1_body(x_ref, w_ref, deg_ref, o_ref):
    d = _dinv(deg_ref)
    p = _dot(x_ref[...], w_ref[...])
    o_ref[0] = p[:, : F1 // 2] * d
    o_ref[1] = p[:, F1 // 2 :] * d


def _tc_matmul1(x, w1, deg2):
    return pl.pallas_call(
        _mm1_body,
        grid=(N // RB,),
        in_specs=[
            pl.BlockSpec((RB, F1), lambda i: (i, 0)),
            pl.BlockSpec((F1, F1), lambda i: (0, 0)),
            pl.BlockSpec((RB, NC), lambda i: (i, 0)),
        ],
        out_specs=pl.BlockSpec((NC, RB, F1 // 2), lambda i: (0, i, 0)),
        out_shape=jax.ShapeDtypeStruct((NC, N, F1 // 2), jnp.float32),
    )(x, w1, deg2)


def _layer2_body(a_ref, deg_ref, b1_ref, w2_ref, o_ref):
    d = _dinv(deg_ref)
    h0 = jnp.maximum(a_ref[0] * d + b1_ref[0, : F1 // 2], 0.0)
    h1 = jnp.maximum(a_ref[1] * d + b1_ref[0, F1 // 2 :], 0.0)
    y = _dot(h0, w2_ref[: F1 // 2, :]) + _dot(h1, w2_ref[F1 // 2 :, :])
    y = y * d
    o_ref[0] = y[:, : F2 // 2]
    o_ref[1] = y[:, F2 // 2 :]


def _tc_layer2(agg1, deg2, b1, w2):
    return pl.pallas_call(
        _layer2_body,
        grid=(N // RB,),
        in_specs=[
            pl.BlockSpec((NC, RB, F1 // 2), lambda i: (0, i, 0)),
            pl.BlockSpec((RB, NC), lambda i: (i, 0)),
            pl.BlockSpec((1, F1), lambda i: (0, 0)),
            pl.BlockSpec((F1, F2), lambda i: (0, 0)),
        ],
        out_specs=pl.BlockSpec((NC, RB, F2 // 2), lambda i: (0, i, 0)),
        out_shape=jax.ShapeDtypeStruct((NC, N, F2 // 2), jnp.float32),
    )(agg1, deg2, b1, w2)


def _final_body(a_ref, deg_ref, b2_ref, o_ref):
    d = _dinv(deg_ref)
    z = jnp.concatenate([a_ref[0], a_ref[1]], axis=1) * d + b2_ref[0, :]
    m = jnp.max(z, axis=1, keepdims=True)
    e = z - m
    lse = jnp.log(jnp.sum(jnp.exp(e), axis=1, keepdims=True))
    o_ref[...] = e - lse


def _tc_final(agg2, deg2, b2):
    return pl.pallas_call(
        _final_body,
        grid=(N // RB,),
        in_specs=[
            pl.BlockSpec((NC, RB, F2 // 2), lambda i: (0, i, 0)),
            pl.BlockSpec((RB, NC), lambda i: (i, 0)),
            pl.BlockSpec((1, F2), lambda i: (0, 0)),
        ],
        out_specs=pl.BlockSpec((RB, F2), lambda i: (i, 0)),
        out_shape=jax.ShapeDtypeStruct((N, F2), jnp.float32),
    )(agg2, deg2, b2)


@jax.jit
def kernel(X, edge_index, W1, b1, W2, b2):
    src = edge_index[0]
    dst = edge_index[1]
    sd1 = _make_sd(src, dst, KE1)
    sd2 = _make_sd(src, dst, KE2)

    deg2 = _sc_degree(dst).reshape(NC, NP)[:, :N].T
    y1 = _tc_matmul1(X, W1, deg2)
    agg1 = _sc_agg1(y1.reshape(NC * N, F1 // 2), sd1).reshape(NC, N, F1 // 2)
    y2 = _tc_layer2(agg1, deg2, b1.reshape(1, F1), W2)
    agg2 = _sc_agg2(y2.reshape(NC * N, F2 // 2), sd2).reshape(NC, N, F2 // 2)
    out = _tc_final(agg2, deg2, b2.reshape(1, F2))
    return out


# speedup vs baseline: 2.3514x; 1.1489x over previous
import functools

import jax
import jax.numpy as jnp
from jax import lax
from jax.experimental import pallas as pl
from jax.experimental.pallas import tpu as pltpu
from jax.experimental.pallas import tpu_sc as plsc

N = 10000
NP = 10240
E = 160000
NC = 2
NS = 16
NW = NC * NS
EW = E // NW
RW = NP // NS
ES = 163840
ET = ES // NS
ACC_R = 10240
RT = N // NS
F1 = 256
F2 = 64
RB = 1000

_mesh = plsc.VectorSubcoreMesh(
    core_axis_name="c", subcore_axis_name="s", num_cores=NC, num_subcores=NS
)

_sc_params = pltpu.CompilerParams(use_tc_tiling_on_sc=False)


@functools.partial(
    pl.kernel,
    out_type=jax.ShapeDtypeStruct((NC * NP,), jnp.float32),
    mesh=_mesh,
    scratch_types=[
        pltpu.VMEM((EW,), jnp.int32),
        pltpu.VMEM((EW + 8,), jnp.float32),
        pltpu.VMEM((RW,), jnp.float32),
        pltpu.VMEM_SHARED((NP,), jnp.float32),
    ],
    compiler_params=_sc_params,
)
def _sc_degree(dst_hbm, out_hbm, idx_v, ones_v, zeros_v, deg_sh):
    c = lax.axis_index("c")
    s = lax.axis_index("s")
    w = s * NC + c

    @pl.loop(0, EW + 8, step=16)
    def _(i):
        ones_v[pl.ds(i, 16)] = jnp.full((16,), 1.0, jnp.float32)

    @pl.loop(0, RW, step=16)
    def _(i):
        zeros_v[pl.ds(i, 16)] = jnp.zeros((16,), jnp.float32)

    pltpu.sync_copy(zeros_v, deg_sh.at[pl.ds(s * RW, RW)])
    plsc.subcore_barrier()
    pltpu.sync_copy(dst_hbm.at[pl.ds(w * EW, EW)], idx_v)
    pltpu.sync_copy(ones_v.at[pl.ds(0, EW)], deg_sh.at[idx_v], add=True)
    plsc.subcore_barrier()
    pltpu.sync_copy(
        deg_sh.at[pl.ds(s * RW, RW)], out_hbm.at[pl.ds(c * NP + s * RW, RW)]
    )


def _make_sc_aggregate(fc, ke):
    nwin = ET // ke
    nwt = ES // ke
    nchp = nwin // 8
    assert nwin % 8 == 0 and ke % 8 == 0 and ET % ke == 0

    @functools.partial(
        pl.kernel,
        out_type=jax.ShapeDtypeStruct((NC * N, fc), jnp.float32),
        mesh=_mesh,
        scratch_types=[
            pltpu.VMEM((8, ke), jnp.int32),
            pltpu.VMEM((8, ke), jnp.int32),
            pltpu.VMEM((ke, fc), jnp.float32),
            pltpu.VMEM((ke, fc), jnp.float32),
            pltpu.VMEM_SHARED((ACC_R, fc), jnp.float32),
            pltpu.SemaphoreType.DMA,
            pltpu.SemaphoreType.DMA,
            pltpu.SemaphoreType.DMA,
            pltpu.SemaphoreType.DMA,
            pltpu.SemaphoreType.DMA,
        ],
        compiler_params=_sc_params,
    )
    def agg(y_hbm, sd_hbm, out_hbm, sd_a, sd_b, rows_a, rows_b, acc_sh,
            sem_la, sem_lb, sem_ga, sem_gb, sem_i):
        c = lax.axis_index("c")
        s = lax.axis_index("s")
        row0 = (c * nwt + s * nwin) * 2

        init = pltpu.async_copy(
            y_hbm.at[pl.ds(c * N + s * RT, RT)],
            acc_sh.at[pl.ds(s * RT, RT)],
            sem_i,
        )
        pltpu.sync_copy(sd_hbm.at[pl.ds(row0, 8), :], sd_a)
        pltpu.async_copy(sd_hbm.at[pl.ds(row0 + 8, 8), :], sd_b, sem_lb)
        pltpu.async_copy(y_hbm.at[sd_a.at[0]], rows_a, sem_ga)
        init.wait()
        plsc.subcore_barrier()

        @pl.loop(0, nchp)
        def _(t):
            for w in range(8):
                cur_sd = sd_a if w < 4 else sd_b
                cur_rows = rows_a if w % 2 == 0 else rows_b
                cur_sem = sem_ga if w % 2 == 0 else sem_gb
                nxt_rows = rows_b if w % 2 == 0 else rows_a
                nxt_sem = sem_gb if w % 2 == 0 else sem_ga
                if w < 7:
                    nxt_sd = sd_a if w + 1 < 4 else sd_b
                    if w + 1 == 4:
                        pltpu.make_async_copy(
                            sd_hbm.at[pl.ds(0, 8), :], sd_b, sem_lb
                        ).wait()
                    pltpu.async_copy(
                        y_hbm.at[nxt_sd.at[2 * ((w + 1) % 4)]], nxt_rows, nxt_sem
                    )
                else:
                    @pl.when(t + 1 < nchp)
                    def _():
                        pltpu.make_async_copy(
                            sd_hbm.at[pl.ds(0, 8), :], sd_a, sem_la
                        ).wait()
                        pltpu.async_copy(y_hbm.at[sd_a.at[0]], nxt_rows, nxt_sem)
                pltpu.make_async_copy(
                    y_hbm.at[pl.ds(0, ke)], cur_rows, cur_sem
                ).wait()
                pltpu.sync_copy(
                    cur_rows, acc_sh.at[cur_sd.at[2 * (w % 4) + 1]], add=True
                )
                if w == 3:
                    @pl.when(t + 1 < nchp)
                    def _():
                        pltpu.async_copy(
                            sd_hbm.at[pl.ds(row0 + (2 * t + 2) * 8, 8), :],
                            sd_a, sem_la,
                        )
                if w == 7:
                    @pl.when(t + 1 < nchp)
                    def _():
                        pltpu.async_copy(
                            sd_hbm.at[pl.ds(row0 + (2 * t + 3) * 8, 8), :],
                            sd_b, sem_lb,
                        )

        plsc.subcore_barrier()
        pltpu.sync_copy(
            acc_sh.at[pl.ds(s * RT, RT)],
            out_hbm.at[pl.ds(c * N + s * RT, RT), :],
        )

    return agg


KE1 = 160
KE2 = 640
_sc_agg1 = _make_sc_aggregate(F1 // 2, KE1)
_sc_agg2 = _make_sc_aggregate(F2 // 2, KE2)


def _make_sd(src, dst, ke):
    pad = jnp.arange(ES - E, dtype=jnp.int32)
    pad_s = (pad * 997) % N
    pad_d = N + pad % (ACC_R - N)
    sw = jnp.concatenate([src, pad_s]).reshape(ES // ke, ke)
    dw = jnp.concatenate([dst, pad_d]).reshape(ES // ke, ke)
    per_core = [
        jnp.stack([sw + c * N, dw], axis=1) for c in range(NC)
    ]
    return jnp.concatenate(per_core).reshape(NC * (ES // ke) * 2, ke)


def _dinv(deg_ref):
    return lax.rsqrt(deg_ref[:, 0] + deg_ref[:, 1] + 1.0)[:, None]


def _dot(a, b):
    return jax.lax.dot(a, b, preferred_element_type=jnp.float32)


def _mm1_body(x_ref, w_ref, deg_ref, o_ref):
    d = _dinv(deg_ref)
    p = _dot(x_ref[...], w_ref[...])
    o_ref[0] = p[:, : F1 // 2] * d
    o_ref[1] = p[:, F1 // 2 :] * d


def _tc_matmul1(x, w1, deg2):
    return pl.pallas_call(
        _mm1_body,
        grid=(N // RB,),
        in_specs=[
            pl.BlockSpec((RB, F1), lambda i: (i, 0)),
            pl.BlockSpec((F1, F1), lambda i: (0, 0)),
            pl.BlockSpec((RB, NC), lambda i: (i, 0)),
        ],
        out_specs=pl.BlockSpec((NC, RB, F1 // 2), lambda i: (0, i, 0)),
        out_shape=jax.ShapeDtypeStruct((NC, N, F1 // 2), jnp.float32),
    )(x, w1, deg2)


def _layer2_body(a_ref, deg_ref, b1_ref, w2_ref, o_ref):
    d = _dinv(deg_ref)
    h0 = jnp.maximum(a_ref[0] * d + b1_ref[0, : F1 // 2], 0.0)
    h1 = jnp.maximum(a_ref[1] * d + b1_ref[0, F1 // 2 :], 0.0)
    y = _dot(h0, w2_ref[: F1 // 2, :]) + _dot(h1, w2_ref[F1 // 2 :, :])
    y = y * d
    o_ref[0] = y[:, : F2 // 2]
    o_ref[1] = y[:, F2 // 2 :]


def _tc_layer2(agg1, deg2, b1, w2):
    return pl.pallas_call(
        _layer2_body,
        grid=(N // RB,),
        in_specs=[
            pl.BlockSpec((NC, RB, F1 // 2), lambda i: (0, i, 0)),
            pl.BlockSpec((RB, NC), lambda i: (i, 0)),
            pl.BlockSpec((1, F1), lambda i: (0, 0)),
            pl.BlockSpec((F1, F2), lambda i: (0, 0)),
        ],
        out_specs=pl.BlockSpec((NC, RB, F2 // 2), lambda i: (0, i, 0)),
        out_shape=jax.ShapeDtypeStruct((NC, N, F2 // 2), jnp.float32),
    )(agg1, deg2, b1, w2)


def _final_body(a_ref, deg_ref, b2_ref, o_ref):
    d = _dinv(deg_ref)
    z = jnp.concatenate([a_ref[0], a_ref[1]], axis=1) * d + b2_ref[0, :]
    m = jnp.max(z, axis=1, keepdims=True)
    e = z - m
    lse = jnp.log(jnp.sum(jnp.exp(e), axis=1, keepdims=True))
    o_ref[...] = e - lse


def _tc_final(agg2, deg2, b2):
    return pl.pallas_call(
        _final_body,
        grid=(N // RB,),
        in_specs=[
            pl.BlockSpec((NC, RB, F2 // 2), lambda i: (0, i, 0)),
            pl.BlockSpec((RB, NC), lambda i: (i, 0)),
            pl.BlockSpec((1, F2), lambda i: (0, 0)),
        ],
        out_specs=pl.BlockSpec((RB, F2), lambda i: (i, 0)),
        out_shape=jax.ShapeDtypeStruct((N, F2), jnp.float32),
    )(agg2, deg2, b2)


@jax.jit
def kernel(X, edge_index, W1, b1, W2, b2):
    src = edge_index[0]
    dst = edge_index[1]
    sd1 = _make_sd(src, dst, KE1)
    sd2 = _make_sd(src, dst, KE2)

    deg2 = _sc_degree(dst).reshape(NC, NP)[:, :N].T
    y1 = _tc_matmul1(X, W1, deg2)
    agg1 = _sc_agg1(y1.reshape(NC * N, F1 // 2), sd1).reshape(NC, N, F1 // 2)
    y2 = _tc_layer2(agg1, deg2, b1.reshape(1, F1), W2)
    agg2 = _sc_agg2(y2.reshape(NC * N, F2 // 2), sd2).reshape(NC, N, F2 // 2)
    out = _tc_final(agg2, deg2, b2.reshape(1, F2))
    return out


# speedup vs baseline: 2.3599x; 1.0036x over previous
import functools

import jax
import jax.numpy as jnp
from jax import lax
from jax.experimental import pallas as pl
from jax.experimental.pallas import tpu as pltpu
from jax.experimental.pallas import tpu_sc as plsc

N = 10000
NP = 10240
E = 160000
NC = 2
NS = 16
NW = NC * NS
EW = E // NW
RW = NP // NS
ES = 163840
ET = ES // NS
ACC_R = 10240
RT = N // NS
F1 = 256
F2 = 64
RB = 1000

_mesh = plsc.VectorSubcoreMesh(
    core_axis_name="c", subcore_axis_name="s", num_cores=NC, num_subcores=NS
)

_sc_params = pltpu.CompilerParams(use_tc_tiling_on_sc=False)


@functools.partial(
    pl.kernel,
    out_type=jax.ShapeDtypeStruct((NC * NP,), jnp.float32),
    mesh=_mesh,
    scratch_types=[
        pltpu.VMEM((EW,), jnp.int32),
        pltpu.VMEM((EW + 8,), jnp.float32),
        pltpu.VMEM((RW,), jnp.float32),
        pltpu.VMEM_SHARED((NP,), jnp.float32),
    ],
    compiler_params=_sc_params,
)
def _sc_degree(ei_hbm, out_hbm, idx_v, ones_v, zeros_v, deg_sh):
    c = lax.axis_index("c")
    s = lax.axis_index("s")
    w = s * NC + c
    dst_hbm = ei_hbm.at[1]

    @pl.loop(0, EW + 8, step=16)
    def _(i):
        ones_v[pl.ds(i, 16)] = jnp.full((16,), 1.0, jnp.float32)

    @pl.loop(0, RW, step=16)
    def _(i):
        zeros_v[pl.ds(i, 16)] = jnp.zeros((16,), jnp.float32)

    pltpu.sync_copy(zeros_v, deg_sh.at[pl.ds(s * RW, RW)])
    plsc.subcore_barrier()
    pltpu.sync_copy(dst_hbm.at[pl.ds(w * EW, EW)], idx_v)
    pltpu.sync_copy(ones_v.at[pl.ds(0, EW)], deg_sh.at[idx_v], add=True)
    plsc.subcore_barrier()
    pltpu.sync_copy(
        deg_sh.at[pl.ds(s * RW, RW)], out_hbm.at[pl.ds(c * NP + s * RW, RW)]
    )


def _make_sc_aggregate(fc, ke):
    nwin = ET // ke
    nwt = ES // ke
    nchp = nwin // 8
    assert nwin % 8 == 0 and ke % 8 == 0 and ET % ke == 0

    @functools.partial(
        pl.kernel,
        out_type=jax.ShapeDtypeStruct((NC * N, fc), jnp.float32),
        mesh=_mesh,
        scratch_types=[
            pltpu.VMEM((8, ke), jnp.int32),
            pltpu.VMEM((8, ke), jnp.int32),
            pltpu.VMEM((ke, fc), jnp.float32),
            pltpu.VMEM((ke, fc), jnp.float32),
            pltpu.VMEM_SHARED((ACC_R, fc), jnp.float32),
            pltpu.SemaphoreType.DMA,
            pltpu.SemaphoreType.DMA,
            pltpu.SemaphoreType.DMA,
            pltpu.SemaphoreType.DMA,
            pltpu.SemaphoreType.DMA,
        ],
        compiler_params=_sc_params,
    )
    def agg(y_hbm, sd_hbm, out_hbm, sd_a, sd_b, rows_a, rows_b, acc_sh,
            sem_la, sem_lb, sem_ga, sem_gb, sem_i):
        c = lax.axis_index("c")
        s = lax.axis_index("s")
        row0 = (c * nwt + s * nwin) * 2

        init = pltpu.async_copy(
            y_hbm.at[pl.ds(c * N + s * RT, RT)],
            acc_sh.at[pl.ds(s * RT, RT)],
            sem_i,
        )
        pltpu.sync_copy(sd_hbm.at[pl.ds(row0, 8), :], sd_a)
        pltpu.async_copy(sd_hbm.at[pl.ds(row0 + 8, 8), :], sd_b, sem_lb)
        pltpu.async_copy(y_hbm.at[sd_a.at[0]], rows_a, sem_ga)
        init.wait()
        plsc.subcore_barrier()

        @pl.loop(0, nchp)
        def _(t):
            for w in range(8):
                cur_sd = sd_a if w < 4 else sd_b
                cur_rows = rows_a if w % 2 == 0 else rows_b
                cur_sem = sem_ga if w % 2 == 0 else sem_gb
                nxt_rows = rows_b if w % 2 == 0 else rows_a
                nxt_sem = sem_gb if w % 2 == 0 else sem_ga
                if w < 7:
                    nxt_sd = sd_a if w + 1 < 4 else sd_b
                    if w + 1 == 4:
                        pltpu.make_async_copy(
                            sd_hbm.at[pl.ds(0, 8), :], sd_b, sem_lb
                        ).wait()
                    pltpu.async_copy(
                        y_hbm.at[nxt_sd.at[2 * ((w + 1) % 4)]], nxt_rows, nxt_sem
                    )
                else:
                    @pl.when(t + 1 < nchp)
                    def _():
                        pltpu.make_async_copy(
                            sd_hbm.at[pl.ds(0, 8), :], sd_a, sem_la
                        ).wait()
                        pltpu.async_copy(y_hbm.at[sd_a.at[0]], nxt_rows, nxt_sem)
                pltpu.make_async_copy(
                    y_hbm.at[pl.ds(0, ke)], cur_rows, cur_sem
                ).wait()
                pltpu.sync_copy(
                    cur_rows, acc_sh.at[cur_sd.at[2 * (w % 4) + 1]], add=True
                )
                if w == 3:
                    @pl.when(t + 1 < nchp)
                    def _():
                        pltpu.async_copy(
                            sd_hbm.at[pl.ds(row0 + (2 * t + 2) * 8, 8), :],
                            sd_a, sem_la,
                        )
                if w == 7:
                    @pl.when(t + 1 < nchp)
                    def _():
                        pltpu.async_copy(
                            sd_hbm.at[pl.ds(row0 + (2 * t + 3) * 8, 8), :],
                            sd_b, sem_lb,
                        )

        plsc.subcore_barrier()
        pltpu.sync_copy(
            acc_sh.at[pl.ds(s * RT, RT)],
            out_hbm.at[pl.ds(c * N + s * RT, RT), :],
        )

    return agg


KE1 = 160
KE2 = 1280
_sc_agg1 = _make_sc_aggregate(F1 // 2, KE1)
_sc_agg2 = _make_sc_aggregate(F2 // 2, KE2)


def _make_sd(src, dst, ke):
    pad = jnp.arange(ES - E, dtype=jnp.int32)
    pad_s = (pad * 997) % N
    pad_d = N + pad % (ACC_R - N)
    sw = jnp.concatenate([src, pad_s]).reshape(ES // ke, ke)
    dw = jnp.concatenate([dst, pad_d]).reshape(ES // ke, ke)
    per_core = [
        jnp.stack([sw + c * N, dw], axis=1) for c in range(NC)
    ]
    return jnp.concatenate(per_core).reshape(NC * (ES // ke) * 2, ke)


def _dinv(deg_ref):
    return lax.rsqrt(deg_ref[:, 0] + deg_ref[:, 1] + 1.0)[:, None]


def _dot(a, b):
    return jax.lax.dot(a, b, preferred_element_type=jnp.float32)


def _mm1_body(x_ref, w_ref, deg_ref, o_ref):
    d = _dinv(deg_ref)
    p = _dot(x_ref[...], w_ref[...])
    o_ref[0] = p[:, : F1 // 2] * d
    o_ref[1] = p[:, F1 // 2 :] * d


def _tc_matmul1(x, w1, deg2):
    return pl.pallas_call(
        _mm1_body,
        grid=(N // RB,),
        in_specs=[
            pl.BlockSpec((RB, F1), lambda i: (i, 0)),
            pl.BlockSpec((F1, F1), lambda i: (0, 0)),
            pl.BlockSpec((RB, NC), lambda i: (i, 0)),
        ],
        out_specs=pl.BlockSpec((NC, RB, F1 // 2), lambda i: (0, i, 0)),
        out_shape=jax.ShapeDtypeStruct((NC, N, F1 // 2), jnp.float32),
    )(x, w1, deg2)


def _layer2_body(a_ref, deg_ref, b1_ref, w2_ref, o_ref):
    d = _dinv(deg_ref)
    h0 = jnp.maximum(a_ref[0] * d + b1_ref[0, : F1 // 2], 0.0)
    h1 = jnp.maximum(a_ref[1] * d + b1_ref[0, F1 // 2 :], 0.0)
    y = _dot(h0, w2_ref[: F1 // 2, :]) + _dot(h1, w2_ref[F1 // 2 :, :])
    y = y * d
    o_ref[0] = y[:, : F2 // 2]
    o_ref[1] = y[:, F2 // 2 :]


def _tc_layer2(agg1, deg2, b1, w2):
    return pl.pallas_call(
        _layer2_body,
        grid=(N // RB,),
        in_specs=[
            pl.BlockSpec((NC, RB, F1 // 2), lambda i: (0, i, 0)),
            pl.BlockSpec((RB, NC), lambda i: (i, 0)),
            pl.BlockSpec((1, F1), lambda i: (0, 0)),
            pl.BlockSpec((F1, F2), lambda i: (0, 0)),
        ],
        out_specs=pl.BlockSpec((NC, RB, F2 // 2), lambda i: (0, i, 0)),
        out_shape=jax.ShapeDtypeStruct((NC, N, F2 // 2), jnp.float32),
    )(agg1, deg2, b1, w2)


def _final_body(a_ref, deg_ref, b2_ref, o_ref):
    d = _dinv(deg_ref)
    z = jnp.concatenate([a_ref[0], a_ref[1]], axis=1) * d + b2_ref[0, :]
    m = jnp.max(z, axis=1, keepdims=True)
    e = z - m
    lse = jnp.log(jnp.sum(jnp.exp(e), axis=1, keepdims=True))
    o_ref[...] = e - lse


def _tc_final(agg2, deg2, b2):
    return pl.pallas_call(
        _final_body,
        grid=(N // RB,),
        in_specs=[
            pl.BlockSpec((NC, RB, F2 // 2), lambda i: (0, i, 0)),
            pl.BlockSpec((RB, NC), lambda i: (i, 0)),
            pl.BlockSpec((1, F2), lambda i: (0, 0)),
        ],
        out_specs=pl.BlockSpec((RB, F2), lambda i: (i, 0)),
        out_shape=jax.ShapeDtypeStruct((N, F2), jnp.float32),
    )(agg2, deg2, b2)


@jax.jit
def kernel(X, edge_index, W1, b1, W2, b2):
    src = edge_index[0]
    dst = edge_index[1]
    sd1 = _make_sd(src, dst, KE1)
    sd2 = _make_sd(src, dst, KE2)

    deg2 = _sc_degree(edge_index).reshape(NC, NP)[:, :N].T
    y1 = _tc_matmul1(X, W1, deg2)
    agg1 = _sc_agg1(y1.reshape(NC * N, F1 // 2), sd1).reshape(NC, N, F1 // 2)
    y2 = _tc_layer2(agg1, deg2, b1.reshape(1, F1), W2)
    agg2 = _sc_agg2(y2.reshape(NC * N, F2 // 2), sd2).reshape(NC, N, F2 // 2)
    out = _tc_final(agg2, deg2, b2.reshape(1, F2))
    return out
